# baseline (speedup 1.0000x reference)
import functools

import jax
import jax.numpy as jnp
from jax import lax
from jax.experimental import pallas as pl
from jax.experimental.pallas import tpu as pltpu
from jax.experimental.pallas import tpu_sc as plsc

EMBED = 64
LANES = 16
NC, NS = 2, 16
NW = NC * NS
CHUNK = 512
IDXW = 128
SUB = CHUNK // IDXW


def _make_kernel(N):
    per_w = N // NW
    n_chunks = per_w // CHUNK
    mesh = plsc.VectorSubcoreMesh(core_axis_name="c", subcore_axis_name="s")

    @functools.partial(
        pl.kernel,
        mesh=mesh,
        out_type=jax.ShapeDtypeStruct((N, EMBED), jnp.float32),
        compiler_params=pltpu.CompilerParams(
            use_tc_tiling_on_sc=False, needs_layout_passes=False),
        scratch_types=[
            pltpu.VMEM((SUB, IDXW), jnp.int32),
            pltpu.VMEM((CHUNK, EMBED), jnp.float32),
            pltpu.VMEM((EMBED, LANES), jnp.float32),
            pltpu.VMEM((EMBED, LANES), jnp.float32),
            pltpu.SemaphoreType.DMA,
        ],
    )
    def k(ids_hbm, table_hbm, gam_hbm, bet_hbm, out_hbm,
          idx_v, rows_v, gam_v, bet_v, sem):
        wid = lax.axis_index("s") * NC + lax.axis_index("c")
        pltpu.sync_copy(gam_hbm, gam_v)
        pltpu.sync_copy(bet_hbm, bet_v)
        lane = lax.iota(jnp.int32, LANES)

        def chunk_body(g, carry):
            row0 = wid * per_w + g * CHUNK
            pltpu.sync_copy(ids_hbm.at[pl.ds(wid * (per_w // IDXW) + g * SUB, SUB)],
                            idx_v)
            cps = [
                pltpu.async_copy(table_hbm.at[idx_v.at[j]],
                                 rows_v.at[pl.ds(j * IDXW, IDXW)], sem)
                for j in range(SUB)
            ]
            for cp in cps:
                cp.wait()

            def group_body(t, carry2):
                ridx = t * LANES + lane
                ssum = jnp.zeros((LANES,), jnp.float32)
                ssq = jnp.zeros((LANES,), jnp.float32)
                for j in range(EMBED):
                    cidx = jnp.full((LANES,), j, jnp.int32)
                    x = plsc.load_gather(rows_v, [ridx, cidx])
                    ssum = ssum + x
                    ssq = ssq + x * x
                mean = ssum * (1.0 / EMBED)
                var = ssq * (1.0 / EMBED) - mean * mean
                v = var + 1e-5
                iv = plsc.bitcast(v, jnp.int32)
                iv = 0x5F3759DF - lax.shift_right_logical(iv, 1)
                y = plsc.bitcast(iv, jnp.float32)
                h = v * 0.5
                for _ in range(3):
                    y = y * (1.5 - h * y * y)
                for j in range(EMBED):
                    cidx = jnp.full((LANES,), j, jnp.int32)
                    x = plsc.load_gather(rows_v, [ridx, cidx])
                    o = (x - mean) * y * gam_v[j] + bet_v[j]
                    plsc.store_scatter(rows_v, [ridx, cidx], o)
                return carry2

            lax.fori_loop(0, CHUNK // LANES, group_body, None)
            pltpu.sync_copy(rows_v, out_hbm.at[pl.ds(row0, CHUNK)])
            return carry

        lax.fori_loop(0, n_chunks, chunk_body, None)

    return k


def kernel(input_ids, table, gamma, beta):
    B, S = input_ids.shape
    _, E = table.shape
    assert E == EMBED
    N = B * S
    assert N % (NW * CHUNK) == 0
    ids = input_ids.reshape(N // IDXW, IDXW).astype(jnp.int32)
    gam = jnp.broadcast_to(gamma.astype(jnp.float32)[:, None], (E, LANES))
    bet = jnp.broadcast_to(beta.astype(jnp.float32)[:, None], (E, LANES))
    out = _make_kernel(N)(ids, table, gam, bet)
    return out.reshape(B, S, E)


# speedup vs baseline: 1.2542x; 1.2542x over previous
import functools

import jax
import jax.numpy as jnp
from jax import lax
from jax.experimental import pallas as pl
from jax.experimental.pallas import tpu as pltpu
from jax.experimental.pallas import tpu_sc as plsc

EMBED = 64
LANES = 16
NC, NS = 2, 16
NW = NC * NS
CHUNK = 512
IDXW = 128
SUB = CHUNK // IDXW
JBLK = 8


def _make_kernel(B, S):
    n_units = S * (B // CHUNK)
    per_w = n_units // NW
    kb_per_s = B // CHUNK
    mesh = plsc.VectorSubcoreMesh(core_axis_name="c", subcore_axis_name="s")

    @functools.partial(
        pl.kernel,
        mesh=mesh,
        out_type=jax.ShapeDtypeStruct((S, EMBED, B), jnp.float32),
        compiler_params=pltpu.CompilerParams(
            use_tc_tiling_on_sc=False, needs_layout_passes=False),
        scratch_types=[
            pltpu.VMEM((SUB, IDXW), jnp.int32),
            pltpu.VMEM((CHUNK, EMBED), jnp.float32),
            pltpu.VMEM((EMBED, CHUNK), jnp.float32),
            pltpu.VMEM((CHUNK,), jnp.float32),
            pltpu.VMEM((CHUNK,), jnp.float32),
            pltpu.VMEM((EMBED, LANES), jnp.float32),
            pltpu.VMEM((EMBED, LANES), jnp.float32),
            pltpu.SemaphoreType.DMA,
        ],
    )
    def k(ids_hbm, table_hbm, gam_hbm, bet_hbm, out_hbm,
          idx_v, rows_v, out_v, sa_v, sc_v, gam_v, bet_v, sem):
        wid = lax.axis_index("s") * NC + lax.axis_index("c")
        pltpu.sync_copy(gam_hbm, gam_v)
        pltpu.sync_copy(bet_hbm, bet_v)
        lane = lax.iota(jnp.int32, LANES)

        def unit_body(i, carry):
            u = wid * per_w + i
            s = u // kb_per_s
            kb = u % kb_per_s
            pltpu.sync_copy(ids_hbm.at[s, pl.ds(kb * SUB, SUB)], idx_v)
            cps = [
                pltpu.async_copy(table_hbm.at[idx_v.at[j]],
                                 rows_v.at[pl.ds(j * IDXW, IDXW)], sem)
                for j in range(SUB)
            ]
            for cp in cps:
                cp.wait()

            def stats_body(t, carry2):
                ridx = t * LANES + lane
                acc = []
                for j in range(EMBED):
                    cidx = jnp.full((LANES,), j, jnp.int32)
                    x = plsc.load_gather(rows_v, [ridx, cidx])
                    acc.append(x)
                ssum = acc[0]
                ssq = acc[0] * acc[0]
                for j in range(1, EMBED):
                    ssum = ssum + acc[j]
                    ssq = ssq + acc[j] * acc[j]
                mean = ssum * (1.0 / EMBED)
                var = ssq * (1.0 / EMBED) - mean * mean
                v = var + 1e-5
                iv = plsc.bitcast(v, jnp.int32)
                iv = 0x5F3759DF - lax.shift_right_logical(iv, 1)
                y = plsc.bitcast(iv, jnp.float32)
                h = v * 0.5
                for _ in range(3):
                    y = y * (1.5 - h * y * y)
                sa_v[pl.ds(t * LANES, LANES)] = y
                sc_v[pl.ds(t * LANES, LANES)] = mean * y
                return carry2

            lax.fori_loop(0, CHUNK // LANES, stats_body, None)

            for jo in range(EMBED // JBLK):
                gs = [gam_v[jo * JBLK + jj] for jj in range(JBLK)]
                bs = [bet_v[jo * JBLK + jj] for jj in range(JBLK)]

                def norm_body(t, carry2, jo=jo, gs=gs, bs=bs):
                    ridx = t * LANES + lane
                    a = sa_v[pl.ds(t * LANES, LANES)]
                    c = sc_v[pl.ds(t * LANES, LANES)]
                    for jj in range(JBLK):
                        j = jo * JBLK + jj
                        cidx = jnp.full((LANES,), j, jnp.int32)
                        x = plsc.load_gather(rows_v, [ridx, cidx])
                        o = (x * a - c) * gs[jj] + bs[jj]
                        out_v[j, pl.ds(t * LANES, LANES)] = o
                    return carry2

                lax.fori_loop(0, CHUNK // LANES, norm_body, None)

            pltpu.sync_copy(out_v,
                            out_hbm.at[s, :, pl.ds(kb * CHUNK, CHUNK)])
            return carry

        lax.fori_loop(0, per_w, unit_body, None)

    return k


def kernel(input_ids, table, gamma, beta):
    B, S = input_ids.shape
    _, E = table.shape
    assert E == EMBED and B % CHUNK == 0 and (S * B // CHUNK) % NW == 0
    ids3 = input_ids.T.astype(jnp.int32).reshape(S, B // IDXW, IDXW)
    gam = jnp.broadcast_to(gamma.astype(jnp.float32)[:, None], (E, LANES))
    bet = jnp.broadcast_to(beta.astype(jnp.float32)[:, None], (E, LANES))
    out = _make_kernel(B, S)(ids3, table, gam, bet)
    return jnp.transpose(out, (2, 0, 1))


# speedup vs baseline: 1.9134x; 1.5256x over previous
import functools

import jax
import jax.numpy as jnp
from jax import lax
from jax.experimental import pallas as pl
from jax.experimental.pallas import tpu as pltpu
from jax.experimental.pallas import tpu_sc as plsc

EMBED = 64
LANES = 16
NC, NS = 2, 16
NW = NC * NS
CHUNK = 256
IDXW = 128
SUB = CHUNK // IDXW
JBLK = 8
GBYTES = IDXW * EMBED * 4
OBYTES = EMBED * CHUNK * 4


def _make_kernel(B, S):
    n_units = S * (B // CHUNK)
    per_w = n_units // NW
    kb_per_s = B // CHUNK
    rows_per_w = per_w * SUB
    mesh = plsc.VectorSubcoreMesh(core_axis_name="c", subcore_axis_name="s")

    @functools.partial(
        pl.kernel,
        mesh=mesh,
        out_type=jax.ShapeDtypeStruct((S, EMBED, B), jnp.float32),
        compiler_params=pltpu.CompilerParams(
            use_tc_tiling_on_sc=False, needs_layout_passes=False),
        scratch_types=[
            pltpu.VMEM((rows_per_w, IDXW), jnp.int32),
            pltpu.VMEM((CHUNK, EMBED), jnp.float32),
            pltpu.VMEM((CHUNK, EMBED), jnp.float32),
            pltpu.VMEM((EMBED, CHUNK), jnp.float32),
            pltpu.VMEM((EMBED, CHUNK), jnp.float32),
            pltpu.VMEM((CHUNK,), jnp.float32),
            pltpu.VMEM((CHUNK,), jnp.float32),
            pltpu.VMEM((EMBED, LANES), jnp.float32),
            pltpu.VMEM((EMBED, LANES), jnp.float32),
            pltpu.SemaphoreType.DMA,
            pltpu.SemaphoreType.DMA,
            pltpu.SemaphoreType.DMA,
            pltpu.SemaphoreType.DMA,
        ],
    )
    def k(ids_hbm, table_hbm, gam_hbm, bet_hbm, out_hbm,
          idx_v, rows_a, rows_b, oa_v, ob_v, sa_v, sc_v, gam_v, bet_v,
          gsa, gsb, osa, osb):
        wid = lax.axis_index("s") * NC + lax.axis_index("c")
        pltpu.sync_copy(gam_hbm, gam_v)
        pltpu.sync_copy(bet_hbm, bet_v)
        pltpu.sync_copy(ids_hbm.at[pl.ds(wid * rows_per_w, rows_per_w)],
                        idx_v)
        lane = lax.iota(jnp.int32, LANES)

        def fire_gathers(c, rows_v, sem):
            for j in range(SUB):
                pltpu.async_copy(table_hbm.at[idx_v.at[c * SUB + j]],
                                 rows_v.at[pl.ds(j * IDXW, IDXW)], sem)

        def wait_gathers(rows_v, sem):
            for j in range(SUB):
                pltpu.make_async_copy(
                    table_hbm.at[idx_v.at[0]],
                    rows_v.at[pl.ds(j * IDXW, IDXW)], sem).wait()

        def out_slab(c):
            u = wid * per_w + c
            s = u // kb_per_s
            kb = u % kb_per_s
            return out_hbm.at[s, :, pl.ds(kb * CHUNK, CHUNK)]

        def compute(rows_v, out_v):
            @plsc.parallel_loop(0, CHUNK // LANES)
            def stats_body(t):
                ridx = t * LANES + lane
                ss = [None] * 4
                qq = [None] * 4
                for j in range(EMBED):
                    cidx = jnp.full((LANES,), j, jnp.int32)
                    x = plsc.load_gather(rows_v, [ridx, cidx])
                    r = j & 3
                    ss[r] = x if ss[r] is None else ss[r] + x
                    qq[r] = x * x if qq[r] is None else qq[r] + x * x
                ssum = (ss[0] + ss[1]) + (ss[2] + ss[3])
                ssq = (qq[0] + qq[1]) + (qq[2] + qq[3])
                mean = ssum * (1.0 / EMBED)
                var = ssq * (1.0 / EMBED) - mean * mean
                v = var + 1e-5
                iv = plsc.bitcast(v, jnp.int32)
                iv = 0x5F3759DF - lax.shift_right_logical(iv, 1)
                y = plsc.bitcast(iv, jnp.float32)
                h = v * 0.5
                y = y * (1.5 - h * y * y)
                y = y * (1.5 - h * y * y)
                sa_v[pl.ds(t * LANES, LANES)] = y
                sc_v[pl.ds(t * LANES, LANES)] = mean * y

            for jo in range(EMBED // JBLK):
                gs = [gam_v[jo * JBLK + jj] for jj in range(JBLK)]
                bs = [bet_v[jo * JBLK + jj] for jj in range(JBLK)]

                @plsc.parallel_loop(0, CHUNK // LANES)
                def norm_body(t, jo=jo, gs=gs, bs=bs):
                    ridx = t * LANES + lane
                    a = sa_v[pl.ds(t * LANES, LANES)]
                    c = sc_v[pl.ds(t * LANES, LANES)]
                    xs = []
                    for jj in range(JBLK):
                        j = jo * JBLK + jj
                        cidx = jnp.full((LANES,), j, jnp.int32)
                        xs.append(plsc.load_gather(rows_v, [ridx, cidx]))
                    os_ = [(xs[jj] * a - c) * gs[jj] + bs[jj]
                           for jj in range(JBLK)]
                    for jj in range(JBLK):
                        out_v[jo * JBLK + jj, pl.ds(t * LANES, LANES)] = os_[jj]

        fire_gathers(0, rows_a, gsa)

        def pair_body(i, carry):
            c0 = 2 * i
            c1 = 2 * i + 1
            c2 = jnp.minimum(2 * i + 2, per_w - 1)
            fire_gathers(c1, rows_b, gsb)
            wait_gathers(rows_a, gsa)

            @pl.when(i > 0)
            def _():
                pltpu.make_async_copy(oa_v, out_slab(0), osa).wait()

            compute(rows_a, oa_v)
            pltpu.async_copy(oa_v, out_slab(c0), osa)
            fire_gathers(c2, rows_a, gsa)
            wait_gathers(rows_b, gsb)

            @pl.when(i > 0)
            def _():
                pltpu.make_async_copy(ob_v, out_slab(0), osb).wait()

            compute(rows_b, ob_v)
            pltpu.async_copy(ob_v, out_slab(c1), osb)
            return carry

        lax.fori_loop(0, per_w // 2, pair_body, None)
        wait_gathers(rows_a, gsa)
        pltpu.make_async_copy(oa_v, out_slab(0), osa).wait()
        pltpu.make_async_copy(ob_v, out_slab(0), osb).wait()

    return k


def kernel(input_ids, table, gamma, beta):
    B, S = input_ids.shape
    _, E = table.shape
    assert E == EMBED and B % CHUNK == 0
    assert (S * B // CHUNK) % (2 * NW) == 0
    ids2 = input_ids.T.astype(jnp.int32).reshape(S * B // IDXW, IDXW)
    gam = jnp.broadcast_to(gamma.astype(jnp.float32)[:, None], (E, LANES))
    bet = jnp.broadcast_to(beta.astype(jnp.float32)[:, None], (E, LANES))
    out = _make_kernel(B, S)(ids2, table, gam, bet)
    return jnp.transpose(out, (2, 0, 1))


# speedup vs baseline: 2.0926x; 1.0937x over previous
import functools

import jax
import jax.numpy as jnp
from jax import lax
from jax.experimental import pallas as pl
from jax.experimental.pallas import tpu as pltpu
from jax.experimental.pallas import tpu_sc as plsc

EMBED = 64
LANES = 16
NC, NS = 2, 16
NW = NC * NS
CHUNK = 128
JBLK = 8
NRING = 4


def _make_kernel(B, S):
    n_units = S * (B // CHUNK)
    per_w = n_units // NW
    kb_per_s = B // CHUNK
    mesh = plsc.VectorSubcoreMesh(core_axis_name="c", subcore_axis_name="s")

    @functools.partial(
        pl.kernel,
        mesh=mesh,
        out_type=jax.ShapeDtypeStruct(
            (S, EMBED // 8, B // CHUNK, 8, CHUNK), jnp.float32),
        compiler_params=pltpu.CompilerParams(
            use_tc_tiling_on_sc=False, needs_layout_passes=False),
        scratch_types=[
            pltpu.VMEM((per_w, CHUNK), jnp.int32),
            *[pltpu.VMEM((CHUNK, EMBED), jnp.float32) for _ in range(NRING)],
            pltpu.VMEM((EMBED // 8, 8, CHUNK), jnp.float32),
            pltpu.VMEM((EMBED // 8, 8, CHUNK), jnp.float32),
            pltpu.VMEM((CHUNK,), jnp.float32),
            pltpu.VMEM((CHUNK,), jnp.float32),
            pltpu.VMEM((EMBED, LANES), jnp.float32),
            pltpu.VMEM((EMBED, LANES), jnp.float32),
            *[pltpu.SemaphoreType.DMA for _ in range(NRING + 2)],
        ],
    )
    def k(ids_hbm, table_hbm, gam_hbm, bet_hbm, out_hbm,
          idx_v, r0, r1, r2, r3, oa_v, ob_v, sa_v, sc_v, gam_v, bet_v,
          g0, g1, g2, g3, osa, osb):
        rows = [r0, r1, r2, r3]
        gsem = [g0, g1, g2, g3]
        outs = [oa_v, ob_v]
        osem = [osa, osb]
        wid = lax.axis_index("s") * NC + lax.axis_index("c")
        pltpu.sync_copy(gam_hbm, gam_v)
        pltpu.sync_copy(bet_hbm, bet_v)
        pltpu.sync_copy(ids_hbm.at[pl.ds(wid * per_w, per_w)], idx_v)
        lane = lax.iota(jnp.int32, LANES)

        def fire_gather(c, r):
            pltpu.async_copy(table_hbm.at[idx_v.at[c]], rows[r], gsem[r])

        def wait_gather(r):
            pltpu.make_async_copy(table_hbm.at[idx_v.at[0]],
                                  rows[r], gsem[r]).wait()

        def fire_out(c, out_v, sem):
            u = wid * per_w + c
            s = u // kb_per_s
            kb = u % kb_per_s
            for eb in range(EMBED // 8):
                pltpu.async_copy(out_v.at[eb], out_hbm.at[s, eb, kb], sem)

        def wait_out(out_v, sem):
            for eb in range(EMBED // 8):
                pltpu.make_async_copy(out_v.at[eb], out_hbm.at[0, eb, 0],
                                      sem).wait()

        def compute(rows_v, out_v):
            @plsc.parallel_loop(0, CHUNK // LANES)
            def stats_body(t):
                ridx = t * LANES + lane
                ss = [None] * 4
                qq = [None] * 4
                for j in range(EMBED):
                    cidx = jnp.full((LANES,), j, jnp.int32)
                    x = plsc.load_gather(rows_v, [ridx, cidx])
                    r = j & 3
                    ss[r] = x if ss[r] is None else ss[r] + x
                    qq[r] = x * x if qq[r] is None else qq[r] + x * x
                ssum = (ss[0] + ss[1]) + (ss[2] + ss[3])
                ssq = (qq[0] + qq[1]) + (qq[2] + qq[3])
                mean = ssum * (1.0 / EMBED)
                var = ssq * (1.0 / EMBED) - mean * mean
                v = var + 1e-5
                iv = plsc.bitcast(v, jnp.int32)
                iv = 0x5F3759DF - lax.shift_right_logical(iv, 1)
                y = plsc.bitcast(iv, jnp.float32)
                h = v * 0.5
                y = y * (1.5 - h * y * y)
                y = y * (1.5 - h * y * y)
                sa_v[pl.ds(t * LANES, LANES)] = y
                sc_v[pl.ds(t * LANES, LANES)] = mean * y

            for jo in range(EMBED // JBLK):
                gs = [gam_v[jo * JBLK + jj] for jj in range(JBLK)]
                bs = [bet_v[jo * JBLK + jj] for jj in range(JBLK)]

                @plsc.parallel_loop(0, CHUNK // LANES)
                def norm_body(t, jo=jo, gs=gs, bs=bs):
                    ridx = t * LANES + lane
                    a = sa_v[pl.ds(t * LANES, LANES)]
                    c = sc_v[pl.ds(t * LANES, LANES)]
                    xs = []
                    for jj in range(JBLK):
                        j = jo * JBLK + jj
                        cidx = jnp.full((LANES,), j, jnp.int32)
                        xs.append(plsc.load_gather(rows_v, [ridx, cidx]))
                    os_ = [(xs[jj] * a - c) * gs[jj] + bs[jj]
                           for jj in range(JBLK)]
                    for jj in range(JBLK):
                        j = jo * JBLK + jj
                        out_v[j // 8, j % 8, pl.ds(t * LANES, LANES)] = os_[jj]

        for r in range(NRING - 1):
            fire_gather(r, r)

        def quad_body(i, carry):
            for q in range(NRING):
                c = NRING * i + q
                fire_gather(jnp.minimum(c + NRING - 1, per_w - 1),
                            (q + NRING - 1) % NRING)
                wait_gather(q)
                op = q & 1
                if q < 2:
                    @pl.when(i > 0)
                    def _():
                        wait_out(outs[op], osem[op])
                else:
                    wait_out(outs[op], osem[op])
                compute(rows[q], outs[op])
                fire_out(c, outs[op], osem[op])
            return carry

        lax.fori_loop(0, per_w // NRING, quad_body, None)
        for r in range(NRING - 1):
            wait_gather(r)
        wait_out(oa_v, osa)
        wait_out(ob_v, osb)

    return k


def kernel(input_ids, table, gamma, beta):
    B, S = input_ids.shape
    _, E = table.shape
    assert E == EMBED and B % CHUNK == 0
    assert (S * B // CHUNK) % (NW * NRING) == 0
    ids2 = input_ids.T.astype(jnp.int32).reshape(S * B // CHUNK, CHUNK)
    gam = jnp.broadcast_to(gamma.astype(jnp.float32)[:, None], (E, LANES))
    bet = jnp.broadcast_to(beta.astype(jnp.float32)[:, None], (E, LANES))
    out5 = _make_kernel(B, S)(ids2, table, gam, bet)
    out = jnp.transpose(out5, (2, 4, 0, 1, 3)).reshape(B, S, E)
    return out
